# Initial kernel scaffold; baseline (speedup 1.0000x reference)
import jax
import jax.numpy as jnp
from jax.experimental import pallas as pl


def kernel(hidden_states, expert_centroids):
    raise NotImplementedError("write your pallas kernel here")


# speedup vs baseline: 4.7950x; 4.7950x over previous
import jax
import jax.numpy as jnp
from jax import lax
from jax.experimental import pallas as pl
from jax.experimental.pallas import tpu as pltpu

_K = 8
_NEG = -3.0e38


def _router_body(x_ref, w_ref, logits_ref, gate_ref, idx_ref):
    x = x_ref[...]
    w = w_ref[...]
    logits = lax.dot_general(
        x, w, dimension_numbers=(((1,), (1,)), ((), ())),
        preferred_element_type=jnp.float32)
    logits_ref[...] = logits

    bm, e_dim = logits.shape
    m = jnp.max(logits, axis=1, keepdims=True)
    ex = jnp.exp(logits - m)
    z = jnp.sum(ex, axis=1, keepdims=True)

    cols = lax.broadcasted_iota(jnp.int32, (bm, e_dim), 1)
    work = logits
    gate = jnp.zeros_like(ex)
    topsum = jnp.zeros((bm, 1), dtype=jnp.float32)
    idx_cols = []
    for _ in range(_K):
        cur = jnp.max(work, axis=1, keepdims=True)
        hit = work == cur
        idxk = jnp.min(jnp.where(hit, cols, e_dim), axis=1, keepdims=True)
        sel = cols == idxk
        gate = gate + jnp.where(sel, ex, 0.0)
        topsum = topsum + jnp.sum(jnp.where(sel, ex, 0.0), axis=1, keepdims=True)
        work = jnp.where(sel, _NEG, work)
        idx_cols.append(idxk)
    denom = topsum + 1e-9 * z
    gate_ref[...] = gate / denom
    idx_ref[...] = jnp.concatenate(idx_cols, axis=1).astype(jnp.int32)


def _route_tc(x2d, centroids):
    n, h = x2d.shape
    e_dim = centroids.shape[0]
    bm = 512
    grid = (n // bm,)
    return pl.pallas_call(
        _router_body,
        grid=grid,
        in_specs=[
            pl.BlockSpec((bm, h), lambda i: (i, 0)),
            pl.BlockSpec((e_dim, h), lambda i: (0, 0)),
        ],
        out_specs=[
            pl.BlockSpec((bm, e_dim), lambda i: (i, 0)),
            pl.BlockSpec((bm, e_dim), lambda i: (i, 0)),
            pl.BlockSpec((bm, _K), lambda i: (i, 0)),
        ],
        out_shape=[
            jax.ShapeDtypeStruct((n, e_dim), jnp.float32),
            jax.ShapeDtypeStruct((n, e_dim), jnp.float32),
            jax.ShapeDtypeStruct((n, _K), jnp.int32),
        ],
        compiler_params=pltpu.CompilerParams(
            dimension_semantics=("parallel",)),
    )(x2d, centroids)


def kernel(hidden_states, expert_centroids):
    b, s, h = hidden_states.shape
    e_dim = expert_centroids.shape[0]
    x2d = hidden_states.reshape(b * s, h)
    logits, gate, idx = _route_tc(x2d, expert_centroids)
    return (gate.reshape(b, s, e_dim),
            idx.reshape(b, s, _K),
            logits.reshape(b, s, e_dim))
